# baseline (speedup 1.0000x reference)
import jax
import jax.numpy as jnp
import numpy as np
from jax import lax
from jax.experimental import pallas as pl
from jax.experimental.pallas import tpu as pltpu
from jax.experimental.pallas import tpu_sc as plsc

D = 16
N = 32768
DIM = 64
L = 4
K = 256
TEMP = 0.1

HN = N // 2
CH = 512
NCH = HN // CH
CAND = K + 16
FMAX = 0x7F800000


def _pair_consts():
    pairs = [(i, i + L) for i in range(D - L)]
    W = np.zeros((len(pairs), 2 * D), np.float32)
    P = np.zeros((len(pairs), 2 * D), np.float32)
    for p, (i, j) in enumerate(pairs):
        cand = [c for c in range(2 * D) if c != i and c != j]
        reps = (K + len(cand) - 1) // len(cand)
        for c in (cand * reps)[:K]:
            W[p, c] += 1.0
        P[p, j] = 1.0
    return W, P, len(pairs)


_W_CONST, _P_CONST, _NPAIRS = _pair_consts()


def _sc_body(xf_hbm, x2_hbm, sums_hbm, stop_hbm,
             chunk, norms, cand_v, cand_i, mrg_v, mrg_i, gsel,
             rows_buf, vec64_a, vec64_b, shared_v, shared_i, sem):
    c = lax.axis_index("c")
    s2 = lax.axis_index("s")
    g = c * (D // 2) + s2 // 2
    h = s2 % 2
    iot = lax.iota(jnp.int32, 16)
    zf = jnp.zeros((16,), jnp.float32)
    zi = jnp.zeros((16,), jnp.int32)
    base_row = g * N + h * HN

    def chunk_body(k, acc):
        pltpu.sync_copy(xf_hbm.at[pl.ds((base_row + k * CH) * DIM, CH * DIM)], chunk)

        def b16_body(b, acc2):
            b0, b1, b2, b3 = acc2
            fbase = (b * 16 + iot) * DIM
            nrm = zf
            for j in range(DIM):
                gv = plsc.load_gather(chunk, [fbase + j])
                nrm = nrm + gv * gv
            row0 = b * 16
            for rr in range(16):
                b0 = b0 + chunk[pl.ds((row0 + rr) * DIM, 16)]
                b1 = b1 + chunk[pl.ds((row0 + rr) * DIM + 16, 16)]
                b2 = b2 + chunk[pl.ds((row0 + rr) * DIM + 32, 16)]
                b3 = b3 + chunk[pl.ds((row0 + rr) * DIM + 48, 16)]
            norms[pl.ds(k * CH + b * 16, 16)] = nrm
            return (b0, b1, b2, b3)

        return lax.fori_loop(0, CH // 16, b16_body, acc)

    a0, a1, a2, a3 = lax.fori_loop(0, NCH, chunk_body, (zf, zf, zf, zf))
    vec64_a[pl.ds(0, 16)] = a0
    vec64_a[pl.ds(16, 16)] = a1
    vec64_a[pl.ds(32, 16)] = a2
    vec64_a[pl.ds(48, 16)] = a3
    pltpu.sync_copy(vec64_a, sums_hbm.at[pl.ds((g * 2 + h) * DIM, DIM)])

    def count_ge(t):
        def cb(i, acc):
            a = acc
            for u in range(8):
                bits = plsc.bitcast(norms[pl.ds(i * 128 + u * 16, 16)], jnp.int32)
                a = a + (bits >= t).astype(jnp.int32)
            return a
        return jnp.sum(lax.fori_loop(0, HN // 128, cb, zi))

    def bis_it(_, lh):
        lo, hi = lh
        mid = lo + (hi - lo) // 2
        take = count_ge(mid) >= K
        return (jnp.where(take, mid, lo), jnp.where(take, hi, mid))

    thr, _ = lax.fori_loop(0, 31, bis_it, (jnp.int32(0), jnp.int32(FMAX)))
    ties_needed = K - count_ge(thr + 1)

    def ebody(i, carry):
        off, tcnt = carry
        v = norms[pl.ds(i * 16, 16)]
        bits = plsc.bitcast(v, jnp.int32)
        m_gt = bits > thr
        m_eq = bits == thr
        pref = plsc.cumsum(m_eq.astype(jnp.int32))
        m_tie = m_eq & ((tcnt + pref) <= ties_needed)
        m_sel = m_gt | m_tie
        plsc.store_compressed(cand_v.at[pl.ds(off, 16)], v, mask=m_sel)
        plsc.store_compressed(cand_i.at[pl.ds(off, 16)],
                              base_row + i * 16 + iot, mask=m_sel)
        return (off + jnp.sum(m_sel.astype(jnp.int32)),
                tcnt + jnp.sum(m_tie.astype(jnp.int32)))

    lax.fori_loop(0, HN // 16, ebody, (jnp.int32(0), jnp.int32(0)))
    cand_v[pl.ds(K, 16)] = jnp.full((16,), -1.0, jnp.float32)
    cand_i[pl.ds(K, 16)] = zi

    pltpu.sync_copy(cand_v, shared_v.at[s2])
    pltpu.sync_copy(cand_i, shared_i.at[s2])
    plsc.subcore_barrier()

    @pl.when(h == 0)
    def _merge():
        pltpu.sync_copy(shared_v.at[s2], mrg_v.at[0])
        pltpu.sync_copy(shared_v.at[s2 + 1], mrg_v.at[1])
        pltpu.sync_copy(shared_i.at[s2], mrg_i.at[0])
        pltpu.sync_copy(shared_i.at[s2 + 1], mrg_i.at[1])

        def mcount(t):
            def cb(i, acc):
                a = acc
                a = a + (plsc.bitcast(mrg_v[0, pl.ds(i * 16, 16)], jnp.int32)
                         >= t).astype(jnp.int32)
                a = a + (plsc.bitcast(mrg_v[1, pl.ds(i * 16, 16)], jnp.int32)
                         >= t).astype(jnp.int32)
                return a
            return jnp.sum(lax.fori_loop(0, CAND // 16, cb, zi))

        def mbis_it(_, lh):
            lo, hi = lh
            mid = lo + (hi - lo) // 2
            take = mcount(mid) >= K
            return (jnp.where(take, mid, lo), jnp.where(take, hi, mid))

        thr2, _ = lax.fori_loop(0, 31, mbis_it, (jnp.int32(0), jnp.int32(FMAX)))
        tneed2 = K - mcount(thr2 + 1)

        def mk_ebody(r):
            def eb(i, carry):
                off, tcnt = carry
                v = mrg_v[r, pl.ds(i * 16, 16)]
                bits = plsc.bitcast(v, jnp.int32)
                m_gt = bits > thr2
                m_eq = bits == thr2
                pref = plsc.cumsum(m_eq.astype(jnp.int32))
                m_tie = m_eq & ((tcnt + pref) <= tneed2)
                m_sel = m_gt | m_tie
                plsc.store_compressed(gsel.at[pl.ds(off, 16)],
                                      mrg_i[r, pl.ds(i * 16, 16)], mask=m_sel)
                return (off + jnp.sum(m_sel.astype(jnp.int32)),
                        tcnt + jnp.sum(m_tie.astype(jnp.int32)))
            return eb

        carry = (jnp.int32(0), jnp.int32(0))
        carry = lax.fori_loop(0, CAND // 16, mk_ebody(0), carry)
        carry = lax.fori_loop(0, CAND // 16, mk_ebody(1), carry)

        WV = 32

        def wave(w, acc):
            def fire(i, _):
                r = gsel[pl.ds(w * WV + i, 16)][0]
                pltpu.async_copy(xf_hbm.at[pl.ds(r * DIM, DIM)],
                                 rows_buf.at[pl.ds(i * DIM, DIM)], sem)
                return 0
            lax.fori_loop(0, WV, fire, 0)
            pltpu.make_async_copy(xf_hbm.at[pl.ds(0, WV * DIM)], rows_buf, sem).wait()

            def rsum(i, acc2):
                b0, b1, b2, b3 = acc2
                b0 = b0 + rows_buf[pl.ds(i * DIM, 16)]
                b1 = b1 + rows_buf[pl.ds(i * DIM + 16, 16)]
                b2 = b2 + rows_buf[pl.ds(i * DIM + 32, 16)]
                b3 = b3 + rows_buf[pl.ds(i * DIM + 48, 16)]
                return (b0, b1, b2, b3)

            return lax.fori_loop(0, WV, rsum, acc)

        st0, st1, st2, st3 = lax.fori_loop(0, K // WV, wave, (zf, zf, zf, zf))
        vec64_b[pl.ds(0, 16)] = st0
        vec64_b[pl.ds(16, 16)] = st1
        vec64_b[pl.ds(32, 16)] = st2
        vec64_b[pl.ds(48, 16)] = st3
        pltpu.sync_copy(vec64_b, stop_hbm.at[pl.ds(g * DIM, DIM)])


def _k4_body(sums_ref, st_ref, w_ref, p_ref, out_ref):
    s2 = sums_ref[...]
    s = s2[:, :DIM] + s2[:, DIM:]
    st = st_ref[...]
    neg = s - st
    samples = jnp.concatenate([s, neg], axis=0)
    nrm = jnp.maximum(jnp.sqrt(jnp.sum(samples * samples, axis=1, keepdims=True)), 1e-8)
    sn = samples / nrm
    G = lax.dot_general(sn, sn, (((1,), (1,)), ((), ())))
    E = jnp.exp(G / TEMP)
    W = w_ref[...]
    P = p_ref[...]
    Ei = E[:_NPAIRS]
    Ej = E[L:L + _NPAIRS]
    Gp = jnp.sum(G[:_NPAIRS] * P, axis=1)
    Epos = jnp.sum(Ei * P, axis=1)
    den_i = Epos + jnp.sum(W * Ei, axis=1)
    den_j = Epos + jnp.sum(W * Ej, axis=1)
    loss = jnp.sum(jnp.log(den_i) + jnp.log(den_j) - 2.0 * Gp / TEMP)
    out_ref[...] = (loss / (_NPAIRS * 2)).reshape(1, 1)


@jax.jit
def kernel(I_embeddings):
    X2 = I_embeddings.reshape(D * N, DIM)

    sums2, stop = pl.kernel(
        _sc_body,
        out_type=[
            jax.ShapeDtypeStruct((D * 2 * DIM,), jnp.float32),
            jax.ShapeDtypeStruct((D * DIM,), jnp.float32),
        ],
        mesh=plsc.VectorSubcoreMesh(core_axis_name="c", subcore_axis_name="s"),
        compiler_params=pltpu.CompilerParams(needs_layout_passes=False),
        scratch_types=[
            pltpu.VMEM((CH * DIM,), jnp.float32),
            pltpu.VMEM((HN,), jnp.float32),
            pltpu.VMEM((CAND,), jnp.float32),
            pltpu.VMEM((CAND,), jnp.int32),
            pltpu.VMEM((2, CAND), jnp.float32),
            pltpu.VMEM((2, CAND), jnp.int32),
            pltpu.VMEM((CAND,), jnp.int32),
            pltpu.VMEM((32 * DIM,), jnp.float32),
            pltpu.VMEM((DIM,), jnp.float32),
            pltpu.VMEM((DIM,), jnp.float32),
            pltpu.VMEM_SHARED((16, CAND), jnp.float32),
            pltpu.VMEM_SHARED((16, CAND), jnp.int32),
            pltpu.SemaphoreType.DMA,
        ],
    )(I_embeddings.reshape(D * N * DIM), X2)

    loss = pl.pallas_call(
        _k4_body,
        grid=(1,),
        in_specs=[
            pl.BlockSpec((D, 2 * DIM), lambda _: (0, 0)),
            pl.BlockSpec((D, DIM), lambda _: (0, 0)),
            pl.BlockSpec((_NPAIRS, 2 * D), lambda _: (0, 0)),
            pl.BlockSpec((_NPAIRS, 2 * D), lambda _: (0, 0)),
        ],
        out_specs=pl.BlockSpec((1, 1), lambda _: (0, 0)),
        out_shape=jax.ShapeDtypeStruct((1, 1), jnp.float32),
    )(sums2.reshape(D, 2 * DIM), stop.reshape(D, DIM),
      jnp.asarray(_W_CONST), jnp.asarray(_P_CONST))

    return loss[0, 0]


# speedup vs baseline: 1.6430x; 1.6430x over previous
import jax
import jax.numpy as jnp
import numpy as np
from jax import lax
from jax.experimental import pallas as pl
from jax.experimental.pallas import tpu as pltpu
from jax.experimental.pallas import tpu_sc as plsc

D = 16
N = 32768
DIM = 64
L = 4
K = 256
TEMP = 0.1

HN = N // 2
CH = 512
NCH = HN // CH
CAND = K + 16
FMAX = 0x7F800000


def _pair_consts():
    pairs = [(i, i + L) for i in range(D - L)]
    W = np.zeros((len(pairs), 2 * D), np.float32)
    P = np.zeros((len(pairs), 2 * D), np.float32)
    for p, (i, j) in enumerate(pairs):
        cand = [c for c in range(2 * D) if c != i and c != j]
        reps = (K + len(cand) - 1) // len(cand)
        for c in (cand * reps)[:K]:
            W[p, c] += 1.0
        P[p, j] = 1.0
    return W, P, len(pairs)


_W_CONST, _P_CONST, _NPAIRS = _pair_consts()


def _sc_body(xf_hbm, sums_hbm, stop_hbm,
             chunk, norms, cand_v, cand_i, mrg_v, mrg_i, gsel,
             rows_buf, vec64_a, vec64_b, shared_v, shared_i, sem):
    c = lax.axis_index("c")
    s2 = lax.axis_index("s")
    g = c * (D // 2) + s2 // 2
    h = s2 % 2
    iot = lax.iota(jnp.int32, 16)
    zf = jnp.zeros((16,), jnp.float32)
    zi = jnp.zeros((16,), jnp.int32)
    base_row = g * N + h * HN

    def chunk_body(k, acc):
        pltpu.sync_copy(xf_hbm.at[pl.ds((base_row + k * CH) * DIM, CH * DIM)], chunk)

        def b16_body(b, acc2):
            b0, b1, b2, b3 = acc2
            nrm = zf
            for j in range(DIM):
                kj = iot * DIM + ((j + iot) & (DIM - 1))
                gv = plsc.load_gather(chunk, [b * (16 * DIM) + kj])
                nrm = nrm + gv * gv
            row0 = b * 16
            for rr in range(16):
                b0 = b0 + chunk[pl.ds((row0 + rr) * DIM, 16)]
                b1 = b1 + chunk[pl.ds((row0 + rr) * DIM + 16, 16)]
                b2 = b2 + chunk[pl.ds((row0 + rr) * DIM + 32, 16)]
                b3 = b3 + chunk[pl.ds((row0 + rr) * DIM + 48, 16)]
            norms[pl.ds(k * CH + b * 16, 16)] = nrm
            return (b0, b1, b2, b3)

        return lax.fori_loop(0, CH // 16, b16_body, acc)

    a0, a1, a2, a3 = lax.fori_loop(0, NCH, chunk_body, (zf, zf, zf, zf))
    vec64_a[pl.ds(0, 16)] = a0
    vec64_a[pl.ds(16, 16)] = a1
    vec64_a[pl.ds(32, 16)] = a2
    vec64_a[pl.ds(48, 16)] = a3
    pltpu.sync_copy(vec64_a, sums_hbm.at[pl.ds((g * 2 + h) * DIM, DIM)])

    def count_ge(t):
        def cb(i, acc):
            a = acc
            for u in range(8):
                bits = plsc.bitcast(norms[pl.ds(i * 128 + u * 16, 16)], jnp.int32)
                a = a + (bits >= t).astype(jnp.int32)
            return a
        return jnp.sum(lax.fori_loop(0, HN // 128, cb, zi))

    def bis_it(_, lh):
        lo, hi = lh
        mid = lo + (hi - lo) // 2
        take = count_ge(mid) >= K
        return (jnp.where(take, mid, lo), jnp.where(take, hi, mid))

    thr, _ = lax.fori_loop(0, 31, bis_it, (jnp.int32(0), jnp.int32(FMAX)))
    ties_needed = K - count_ge(thr + 1)

    def ebody(i, carry):
        off, tcnt = carry
        v = norms[pl.ds(i * 16, 16)]
        bits = plsc.bitcast(v, jnp.int32)
        m_gt = bits > thr
        m_eq = bits == thr
        pref = plsc.cumsum(m_eq.astype(jnp.int32))
        m_tie = m_eq & ((tcnt + pref) <= ties_needed)
        m_sel = m_gt | m_tie
        plsc.store_compressed(cand_v.at[pl.ds(off, 16)], v, mask=m_sel)
        plsc.store_compressed(cand_i.at[pl.ds(off, 16)],
                              base_row + i * 16 + iot, mask=m_sel)
        return (off + jnp.sum(m_sel.astype(jnp.int32)),
                tcnt + jnp.sum(m_tie.astype(jnp.int32)))

    lax.fori_loop(0, HN // 16, ebody, (jnp.int32(0), jnp.int32(0)))
    cand_v[pl.ds(K, 16)] = jnp.full((16,), -1.0, jnp.float32)
    cand_i[pl.ds(K, 16)] = zi

    pltpu.sync_copy(cand_v, shared_v.at[s2])
    pltpu.sync_copy(cand_i, shared_i.at[s2])
    plsc.subcore_barrier()

    @pl.when(h == 0)
    def _merge():
        pltpu.sync_copy(shared_v.at[s2], mrg_v.at[0])
        pltpu.sync_copy(shared_v.at[s2 + 1], mrg_v.at[1])
        pltpu.sync_copy(shared_i.at[s2], mrg_i.at[0])
        pltpu.sync_copy(shared_i.at[s2 + 1], mrg_i.at[1])

        def mcount(t):
            def cb(i, acc):
                a = acc
                a = a + (plsc.bitcast(mrg_v[0, pl.ds(i * 16, 16)], jnp.int32)
                         >= t).astype(jnp.int32)
                a = a + (plsc.bitcast(mrg_v[1, pl.ds(i * 16, 16)], jnp.int32)
                         >= t).astype(jnp.int32)
                return a
            return jnp.sum(lax.fori_loop(0, CAND // 16, cb, zi))

        def mbis_it(_, lh):
            lo, hi = lh
            mid = lo + (hi - lo) // 2
            take = mcount(mid) >= K
            return (jnp.where(take, mid, lo), jnp.where(take, hi, mid))

        thr2, _ = lax.fori_loop(0, 31, mbis_it, (jnp.int32(0), jnp.int32(FMAX)))
        tneed2 = K - mcount(thr2 + 1)

        def mk_ebody(r):
            def eb(i, carry):
                off, tcnt = carry
                v = mrg_v[r, pl.ds(i * 16, 16)]
                bits = plsc.bitcast(v, jnp.int32)
                m_gt = bits > thr2
                m_eq = bits == thr2
                pref = plsc.cumsum(m_eq.astype(jnp.int32))
                m_tie = m_eq & ((tcnt + pref) <= tneed2)
                m_sel = m_gt | m_tie
                plsc.store_compressed(gsel.at[pl.ds(off, 16)],
                                      mrg_i[r, pl.ds(i * 16, 16)], mask=m_sel)
                return (off + jnp.sum(m_sel.astype(jnp.int32)),
                        tcnt + jnp.sum(m_tie.astype(jnp.int32)))
            return eb

        carry = (jnp.int32(0), jnp.int32(0))
        carry = lax.fori_loop(0, CAND // 16, mk_ebody(0), carry)
        carry = lax.fori_loop(0, CAND // 16, mk_ebody(1), carry)

        WV = 32

        def wave(w, acc):
            def fire(i, _):
                r = gsel[pl.ds(w * WV + i, 16)][0]
                pltpu.async_copy(xf_hbm.at[pl.ds(r * DIM, DIM)],
                                 rows_buf.at[pl.ds(i * DIM, DIM)], sem)
                return 0
            lax.fori_loop(0, WV, fire, 0)
            pltpu.make_async_copy(xf_hbm.at[pl.ds(0, WV * DIM)], rows_buf, sem).wait()

            def rsum(i, acc2):
                b0, b1, b2, b3 = acc2
                b0 = b0 + rows_buf[pl.ds(i * DIM, 16)]
                b1 = b1 + rows_buf[pl.ds(i * DIM + 16, 16)]
                b2 = b2 + rows_buf[pl.ds(i * DIM + 32, 16)]
                b3 = b3 + rows_buf[pl.ds(i * DIM + 48, 16)]
                return (b0, b1, b2, b3)

            return lax.fori_loop(0, WV, rsum, acc)

        st0, st1, st2, st3 = lax.fori_loop(0, K // WV, wave, (zf, zf, zf, zf))
        vec64_b[pl.ds(0, 16)] = st0
        vec64_b[pl.ds(16, 16)] = st1
        vec64_b[pl.ds(32, 16)] = st2
        vec64_b[pl.ds(48, 16)] = st3
        pltpu.sync_copy(vec64_b, stop_hbm.at[pl.ds(g * DIM, DIM)])


def _k4_body(sums_ref, st_ref, w_ref, p_ref, out_ref):
    s2 = sums_ref[...]
    s = s2[:, :DIM] + s2[:, DIM:]
    st = st_ref[...]
    neg = s - st
    samples = jnp.concatenate([s, neg], axis=0)
    nrm = jnp.maximum(jnp.sqrt(jnp.sum(samples * samples, axis=1, keepdims=True)), 1e-8)
    sn = samples / nrm
    G = lax.dot_general(sn, sn, (((1,), (1,)), ((), ())))
    E = jnp.exp(G / TEMP)
    W = w_ref[...]
    P = p_ref[...]
    Ei = E[:_NPAIRS]
    Ej = E[L:L + _NPAIRS]
    Gp = jnp.sum(G[:_NPAIRS] * P, axis=1)
    Epos = jnp.sum(Ei * P, axis=1)
    den_i = Epos + jnp.sum(W * Ei, axis=1)
    den_j = Epos + jnp.sum(W * Ej, axis=1)
    loss = jnp.sum(jnp.log(den_i) + jnp.log(den_j) - 2.0 * Gp / TEMP)
    out_ref[...] = (loss / (_NPAIRS * 2)).reshape(1, 1)


@jax.jit
def kernel(I_embeddings):
    sums2, stop = pl.kernel(
        _sc_body,
        out_type=[
            jax.ShapeDtypeStruct((D * 2 * DIM,), jnp.float32),
            jax.ShapeDtypeStruct((D * DIM,), jnp.float32),
        ],
        mesh=plsc.VectorSubcoreMesh(core_axis_name="c", subcore_axis_name="s"),
        compiler_params=pltpu.CompilerParams(needs_layout_passes=False),
        scratch_types=[
            pltpu.VMEM((CH * DIM,), jnp.float32),
            pltpu.VMEM((HN,), jnp.float32),
            pltpu.VMEM((CAND,), jnp.float32),
            pltpu.VMEM((CAND,), jnp.int32),
            pltpu.VMEM((2, CAND), jnp.float32),
            pltpu.VMEM((2, CAND), jnp.int32),
            pltpu.VMEM((CAND,), jnp.int32),
            pltpu.VMEM((32 * DIM,), jnp.float32),
            pltpu.VMEM((DIM,), jnp.float32),
            pltpu.VMEM((DIM,), jnp.float32),
            pltpu.VMEM_SHARED((16, CAND), jnp.float32),
            pltpu.VMEM_SHARED((16, CAND), jnp.int32),
            pltpu.SemaphoreType.DMA,
        ],
    )(I_embeddings.reshape(D * N * DIM))

    loss = pl.pallas_call(
        _k4_body,
        grid=(1,),
        in_specs=[
            pl.BlockSpec((D, 2 * DIM), lambda _: (0, 0)),
            pl.BlockSpec((D, DIM), lambda _: (0, 0)),
            pl.BlockSpec((_NPAIRS, 2 * D), lambda _: (0, 0)),
            pl.BlockSpec((_NPAIRS, 2 * D), lambda _: (0, 0)),
        ],
        out_specs=pl.BlockSpec((1, 1), lambda _: (0, 0)),
        out_shape=jax.ShapeDtypeStruct((1, 1), jnp.float32),
    )(sums2.reshape(D, 2 * DIM), stop.reshape(D, DIM),
      jnp.asarray(_W_CONST), jnp.asarray(_P_CONST))

    return loss[0, 0]


# speedup vs baseline: 1.8991x; 1.1559x over previous
import jax
import jax.numpy as jnp
import numpy as np
from jax import lax
from jax.experimental import pallas as pl
from jax.experimental.pallas import tpu as pltpu
from jax.experimental.pallas import tpu_sc as plsc

D = 16
N = 32768
DIM = 64
L = 4
K = 256
TEMP = 0.1

HN = N // 2
CH = 512
NCH = HN // CH
CAND = K + 16
FMAX = 0x7F800000


def _pair_consts():
    pairs = [(i, i + L) for i in range(D - L)]
    W = np.zeros((len(pairs), 2 * D), np.float32)
    P = np.zeros((len(pairs), 2 * D), np.float32)
    for p, (i, j) in enumerate(pairs):
        cand = [c for c in range(2 * D) if c != i and c != j]
        reps = (K + len(cand) - 1) // len(cand)
        for c in (cand * reps)[:K]:
            W[p, c] += 1.0
        P[p, j] = 1.0
    return W, P, len(pairs)


_W_CONST, _P_CONST, _NPAIRS = _pair_consts()


def _sc_body(xf_hbm, sums_hbm, stop_hbm,
             chunk, norms, cand_v, cand_i, mrg_v, mrg_i, gsel,
             rows_buf, vec64_a, vec64_b, shared_v, shared_i, sem):
    c = lax.axis_index("c")
    s2 = lax.axis_index("s")
    g = c * (D // 2) + s2 // 2
    h = s2 % 2
    iot = lax.iota(jnp.int32, 16)
    zf = jnp.zeros((16,), jnp.float32)
    zi = jnp.zeros((16,), jnp.int32)
    base_row = g * N + h * HN

    def chunk_body(k, acc):
        pltpu.sync_copy(xf_hbm.at[g, pl.ds(h * HN + k * CH, CH), :], chunk)

        def b16_body(b, acc2):
            b0, b1, b2, b3 = acc2
            nrm = zf
            rows16 = b * 16 + iot
            for j in range(DIM):
                cols16 = (j + iot) & (DIM - 1)
                gv = plsc.load_gather(chunk, [rows16, cols16])
                nrm = nrm + gv * gv
            row0 = b * 16
            for rr in range(16):
                b0 = b0 + chunk[row0 + rr, pl.ds(0, 16)]
                b1 = b1 + chunk[row0 + rr, pl.ds(16, 16)]
                b2 = b2 + chunk[row0 + rr, pl.ds(32, 16)]
                b3 = b3 + chunk[row0 + rr, pl.ds(48, 16)]
            norms[pl.ds(k * CH + b * 16, 16)] = nrm
            return (b0, b1, b2, b3)

        return lax.fori_loop(0, CH // 16, b16_body, acc)

    a0, a1, a2, a3 = lax.fori_loop(0, NCH, chunk_body, (zf, zf, zf, zf))
    vec64_a[pl.ds(0, 16)] = a0
    vec64_a[pl.ds(16, 16)] = a1
    vec64_a[pl.ds(32, 16)] = a2
    vec64_a[pl.ds(48, 16)] = a3
    pltpu.sync_copy(vec64_a, sums_hbm.at[pl.ds((g * 2 + h) * DIM, DIM)])

    def count_ge(t):
        def cb(i, acc):
            a = acc
            for u in range(8):
                bits = plsc.bitcast(norms[pl.ds(i * 128 + u * 16, 16)], jnp.int32)
                a = a + (bits >= t).astype(jnp.int32)
            return a
        return jnp.sum(lax.fori_loop(0, HN // 128, cb, zi))

    def bis_it(_, lh):
        lo, hi = lh
        mid = lo + (hi - lo) // 2
        take = count_ge(mid) >= K
        return (jnp.where(take, mid, lo), jnp.where(take, hi, mid))

    thr, _ = lax.fori_loop(0, 31, bis_it, (jnp.int32(0), jnp.int32(FMAX)))
    ties_needed = K - count_ge(thr + 1)

    def ebody(i, carry):
        off, tcnt = carry
        v = norms[pl.ds(i * 16, 16)]
        bits = plsc.bitcast(v, jnp.int32)
        m_gt = bits > thr
        m_eq = bits == thr
        pref = plsc.cumsum(m_eq.astype(jnp.int32))
        m_tie = m_eq & ((tcnt + pref) <= ties_needed)
        m_sel = m_gt | m_tie
        plsc.store_compressed(cand_v.at[pl.ds(off, 16)], v, mask=m_sel)
        plsc.store_compressed(cand_i.at[pl.ds(off, 16)],
                              h * HN + i * 16 + iot, mask=m_sel)
        return (off + jnp.sum(m_sel.astype(jnp.int32)),
                tcnt + jnp.sum(m_tie.astype(jnp.int32)))

    lax.fori_loop(0, HN // 16, ebody, (jnp.int32(0), jnp.int32(0)))
    cand_v[pl.ds(K, 16)] = jnp.full((16,), -1.0, jnp.float32)
    cand_i[pl.ds(K, 16)] = zi

    pltpu.sync_copy(cand_v, shared_v.at[s2])
    pltpu.sync_copy(cand_i, shared_i.at[s2])
    plsc.subcore_barrier()

    @pl.when(h == 0)
    def _merge():
        pltpu.sync_copy(shared_v.at[s2], mrg_v.at[0])
        pltpu.sync_copy(shared_v.at[s2 + 1], mrg_v.at[1])
        pltpu.sync_copy(shared_i.at[s2], mrg_i.at[0])
        pltpu.sync_copy(shared_i.at[s2 + 1], mrg_i.at[1])

        def mcount(t):
            def cb(i, acc):
                a = acc
                a = a + (plsc.bitcast(mrg_v[0, pl.ds(i * 16, 16)], jnp.int32)
                         >= t).astype(jnp.int32)
                a = a + (plsc.bitcast(mrg_v[1, pl.ds(i * 16, 16)], jnp.int32)
                         >= t).astype(jnp.int32)
                return a
            return jnp.sum(lax.fori_loop(0, CAND // 16, cb, zi))

        def mbis_it(_, lh):
            lo, hi = lh
            mid = lo + (hi - lo) // 2
            take = mcount(mid) >= K
            return (jnp.where(take, mid, lo), jnp.where(take, hi, mid))

        thr2, _ = lax.fori_loop(0, 31, mbis_it, (jnp.int32(0), jnp.int32(FMAX)))
        tneed2 = K - mcount(thr2 + 1)

        def mk_ebody(r):
            def eb(i, carry):
                off, tcnt = carry
                v = mrg_v[r, pl.ds(i * 16, 16)]
                bits = plsc.bitcast(v, jnp.int32)
                m_gt = bits > thr2
                m_eq = bits == thr2
                pref = plsc.cumsum(m_eq.astype(jnp.int32))
                m_tie = m_eq & ((tcnt + pref) <= tneed2)
                m_sel = m_gt | m_tie
                plsc.store_compressed(gsel.at[pl.ds(off, 16)],
                                      mrg_i[r, pl.ds(i * 16, 16)], mask=m_sel)
                return (off + jnp.sum(m_sel.astype(jnp.int32)),
                        tcnt + jnp.sum(m_tie.astype(jnp.int32)))
            return eb

        carry = (jnp.int32(0), jnp.int32(0))
        carry = lax.fori_loop(0, CAND // 16, mk_ebody(0), carry)
        carry = lax.fori_loop(0, CAND // 16, mk_ebody(1), carry)

        WV = 32

        def wave(w, acc):
            def fire(i, _):
                r = gsel[pl.ds(w * WV + i, 16)][0]
                pltpu.async_copy(xf_hbm.at[g, r, :], rows_buf.at[i], sem)
                return 0
            lax.fori_loop(0, WV, fire, 0)
            pltpu.make_async_copy(xf_hbm.at[g, pl.ds(0, WV), :], rows_buf, sem).wait()

            def rsum(i, acc2):
                b0, b1, b2, b3 = acc2
                b0 = b0 + rows_buf[i, pl.ds(0, 16)]
                b1 = b1 + rows_buf[i, pl.ds(16, 16)]
                b2 = b2 + rows_buf[i, pl.ds(32, 16)]
                b3 = b3 + rows_buf[i, pl.ds(48, 16)]
                return (b0, b1, b2, b3)

            return lax.fori_loop(0, WV, rsum, acc)

        st0, st1, st2, st3 = lax.fori_loop(0, K // WV, wave, (zf, zf, zf, zf))
        vec64_b[pl.ds(0, 16)] = st0
        vec64_b[pl.ds(16, 16)] = st1
        vec64_b[pl.ds(32, 16)] = st2
        vec64_b[pl.ds(48, 16)] = st3
        pltpu.sync_copy(vec64_b, stop_hbm.at[pl.ds(g * DIM, DIM)])


def _k4_body(sums_ref, st_ref, w_ref, p_ref, out_ref):
    s2 = sums_ref[...]
    s = s2[:, :DIM] + s2[:, DIM:]
    st = st_ref[...]
    neg = s - st
    samples = jnp.concatenate([s, neg], axis=0)
    nrm = jnp.maximum(jnp.sqrt(jnp.sum(samples * samples, axis=1, keepdims=True)), 1e-8)
    sn = samples / nrm
    G = lax.dot_general(sn, sn, (((1,), (1,)), ((), ())))
    E = jnp.exp(G / TEMP)
    W = w_ref[...]
    P = p_ref[...]
    Ei = E[:_NPAIRS]
    Ej = E[L:L + _NPAIRS]
    Gp = jnp.sum(G[:_NPAIRS] * P, axis=1)
    Epos = jnp.sum(Ei * P, axis=1)
    den_i = Epos + jnp.sum(W * Ei, axis=1)
    den_j = Epos + jnp.sum(W * Ej, axis=1)
    loss = jnp.sum(jnp.log(den_i) + jnp.log(den_j) - 2.0 * Gp / TEMP)
    out_ref[...] = (loss / (_NPAIRS * 2)).reshape(1, 1)


@jax.jit
def kernel(I_embeddings):
    sums2, stop = pl.kernel(
        _sc_body,
        out_type=[
            jax.ShapeDtypeStruct((D * 2 * DIM,), jnp.float32),
            jax.ShapeDtypeStruct((D * DIM,), jnp.float32),
        ],
        mesh=plsc.VectorSubcoreMesh(core_axis_name="c", subcore_axis_name="s"),
        compiler_params=pltpu.CompilerParams(needs_layout_passes=False),
        scratch_types=[
            pltpu.VMEM((CH, DIM), jnp.float32),
            pltpu.VMEM((HN,), jnp.float32),
            pltpu.VMEM((CAND,), jnp.float32),
            pltpu.VMEM((CAND,), jnp.int32),
            pltpu.VMEM((2, CAND), jnp.float32),
            pltpu.VMEM((2, CAND), jnp.int32),
            pltpu.VMEM((CAND,), jnp.int32),
            pltpu.VMEM((32, DIM), jnp.float32),
            pltpu.VMEM((DIM,), jnp.float32),
            pltpu.VMEM((DIM,), jnp.float32),
            pltpu.VMEM_SHARED((16, CAND), jnp.float32),
            pltpu.VMEM_SHARED((16, CAND), jnp.int32),
            pltpu.SemaphoreType.DMA,
        ],
    )(I_embeddings)

    loss = pl.pallas_call(
        _k4_body,
        grid=(1,),
        in_specs=[
            pl.BlockSpec((D, 2 * DIM), lambda _: (0, 0)),
            pl.BlockSpec((D, DIM), lambda _: (0, 0)),
            pl.BlockSpec((_NPAIRS, 2 * D), lambda _: (0, 0)),
            pl.BlockSpec((_NPAIRS, 2 * D), lambda _: (0, 0)),
        ],
        out_specs=pl.BlockSpec((1, 1), lambda _: (0, 0)),
        out_shape=jax.ShapeDtypeStruct((1, 1), jnp.float32),
    )(sums2.reshape(D, 2 * DIM), stop.reshape(D, DIM),
      jnp.asarray(_W_CONST), jnp.asarray(_P_CONST))

    return loss[0, 0]


# speedup vs baseline: 2.0197x; 1.0635x over previous
import jax
import jax.numpy as jnp
import numpy as np
from jax import lax
from jax.experimental import pallas as pl
from jax.experimental.pallas import tpu as pltpu
from jax.experimental.pallas import tpu_sc as plsc

D = 16
N = 32768
DIM = 64
L = 4
K = 256
TEMP = 0.1

HN = N // 2
CH = 256
NCH = HN // CH
CAND = K + 16
FMAX = 0x7F800000


def _pair_consts():
    pairs = [(i, i + L) for i in range(D - L)]
    W = np.zeros((len(pairs), 2 * D), np.float32)
    P = np.zeros((len(pairs), 2 * D), np.float32)
    for p, (i, j) in enumerate(pairs):
        cand = [c for c in range(2 * D) if c != i and c != j]
        reps = (K + len(cand) - 1) // len(cand)
        for c in (cand * reps)[:K]:
            W[p, c] += 1.0
        P[p, j] = 1.0
    return W, P, len(pairs)


_W_CONST, _P_CONST, _NPAIRS = _pair_consts()


def _sc_body(xf_hbm, sums_hbm, stop_hbm,
             chunk, chunk_b, norms, cand_v, cand_i, mrg_v, mrg_i, gsel,
             rows_buf, vec64_a, vec64_b, shared_v, shared_i, sem, sem_b):
    c = lax.axis_index("c")
    s2 = lax.axis_index("s")
    g = c * (D // 2) + s2 // 2
    h = s2 % 2
    iot = lax.iota(jnp.int32, 16)
    zf = jnp.zeros((16,), jnp.float32)
    zi = jnp.zeros((16,), jnp.int32)
    base_row = g * N + h * HN

    def fire(buf, sm, k):
        pltpu.async_copy(xf_hbm.at[g, pl.ds(h * HN + k * CH, CH), :], buf, sm)

    def drain(buf, sm):
        pltpu.make_async_copy(xf_hbm.at[g, pl.ds(0, CH), :], buf, sm).wait()

    def compute(buf, k, acc):
        def b16_body(b, acc2):
            b0, b1, b2, b3 = acc2
            nrm = zf
            rows16 = b * 16 + iot
            for j in range(DIM):
                cols16 = (j + iot) & (DIM - 1)
                gv = plsc.load_gather(buf, [rows16, cols16])
                nrm = nrm + gv * gv
            row0 = b * 16
            for rr in range(16):
                b0 = b0 + buf[row0 + rr, pl.ds(0, 16)]
                b1 = b1 + buf[row0 + rr, pl.ds(16, 16)]
                b2 = b2 + buf[row0 + rr, pl.ds(32, 16)]
                b3 = b3 + buf[row0 + rr, pl.ds(48, 16)]
            norms[pl.ds(k * CH + b * 16, 16)] = nrm
            return (b0, b1, b2, b3)

        return lax.fori_loop(0, CH // 16, b16_body, acc)

    fire(chunk, sem, 0)

    def pair_body(k2, acc):
        ka = 2 * k2
        drain(chunk, sem)
        fire(chunk_b, sem_b, ka + 1)
        acc = compute(chunk, ka, acc)
        drain(chunk_b, sem_b)
        acc = compute(chunk_b, ka + 1, acc)

        @pl.when(ka + 2 < NCH)
        def _():
            fire(chunk, sem, ka + 2)

        return acc

    a0, a1, a2, a3 = lax.fori_loop(0, NCH // 2, pair_body, (zf, zf, zf, zf))
    vec64_a[pl.ds(0, 16)] = a0
    vec64_a[pl.ds(16, 16)] = a1
    vec64_a[pl.ds(32, 16)] = a2
    vec64_a[pl.ds(48, 16)] = a3
    pltpu.sync_copy(vec64_a, sums_hbm.at[pl.ds((g * 2 + h) * DIM, DIM)])

    def count_ge(t):
        def cb(i, acc):
            a = acc
            for u in range(8):
                bits = plsc.bitcast(norms[pl.ds(i * 128 + u * 16, 16)], jnp.int32)
                a = a + (bits >= t).astype(jnp.int32)
            return a
        return jnp.sum(lax.fori_loop(0, HN // 128, cb, zi))

    def bis_it(_, lh):
        lo, hi = lh
        mid = lo + (hi - lo) // 2
        take = count_ge(mid) >= K
        return (jnp.where(take, mid, lo), jnp.where(take, hi, mid))

    thr, _ = lax.fori_loop(0, 31, bis_it, (jnp.int32(0), jnp.int32(FMAX)))
    ties_needed = K - count_ge(thr + 1)

    def ebody(i, carry):
        off, tcnt = carry
        v = norms[pl.ds(i * 16, 16)]
        bits = plsc.bitcast(v, jnp.int32)
        m_gt = bits > thr
        m_eq = bits == thr
        pref = plsc.cumsum(m_eq.astype(jnp.int32))
        m_tie = m_eq & ((tcnt + pref) <= ties_needed)
        m_sel = m_gt | m_tie
        plsc.store_compressed(cand_v.at[pl.ds(off, 16)], v, mask=m_sel)
        plsc.store_compressed(cand_i.at[pl.ds(off, 16)],
                              h * HN + i * 16 + iot, mask=m_sel)
        return (off + jnp.sum(m_sel.astype(jnp.int32)),
                tcnt + jnp.sum(m_tie.astype(jnp.int32)))

    lax.fori_loop(0, HN // 16, ebody, (jnp.int32(0), jnp.int32(0)))
    cand_v[pl.ds(K, 16)] = jnp.full((16,), -1.0, jnp.float32)
    cand_i[pl.ds(K, 16)] = zi

    pltpu.sync_copy(cand_v, shared_v.at[s2])
    pltpu.sync_copy(cand_i, shared_i.at[s2])
    plsc.subcore_barrier()

    @pl.when(h == 0)
    def _merge():
        pltpu.sync_copy(shared_v.at[s2], mrg_v.at[0])
        pltpu.sync_copy(shared_v.at[s2 + 1], mrg_v.at[1])
        pltpu.sync_copy(shared_i.at[s2], mrg_i.at[0])
        pltpu.sync_copy(shared_i.at[s2 + 1], mrg_i.at[1])

        def mcount(t):
            def cb(i, acc):
                a = acc
                a = a + (plsc.bitcast(mrg_v[0, pl.ds(i * 16, 16)], jnp.int32)
                         >= t).astype(jnp.int32)
                a = a + (plsc.bitcast(mrg_v[1, pl.ds(i * 16, 16)], jnp.int32)
                         >= t).astype(jnp.int32)
                return a
            return jnp.sum(lax.fori_loop(0, CAND // 16, cb, zi))

        def mbis_it(_, lh):
            lo, hi = lh
            mid = lo + (hi - lo) // 2
            take = mcount(mid) >= K
            return (jnp.where(take, mid, lo), jnp.where(take, hi, mid))

        thr2, _ = lax.fori_loop(0, 31, mbis_it, (jnp.int32(0), jnp.int32(FMAX)))
        tneed2 = K - mcount(thr2 + 1)

        def mk_ebody(r):
            def eb(i, carry):
                off, tcnt = carry
                v = mrg_v[r, pl.ds(i * 16, 16)]
                bits = plsc.bitcast(v, jnp.int32)
                m_gt = bits > thr2
                m_eq = bits == thr2
                pref = plsc.cumsum(m_eq.astype(jnp.int32))
                m_tie = m_eq & ((tcnt + pref) <= tneed2)
                m_sel = m_gt | m_tie
                plsc.store_compressed(gsel.at[pl.ds(off, 16)],
                                      mrg_i[r, pl.ds(i * 16, 16)], mask=m_sel)
                return (off + jnp.sum(m_sel.astype(jnp.int32)),
                        tcnt + jnp.sum(m_tie.astype(jnp.int32)))
            return eb

        carry = (jnp.int32(0), jnp.int32(0))
        carry = lax.fori_loop(0, CAND // 16, mk_ebody(0), carry)
        carry = lax.fori_loop(0, CAND // 16, mk_ebody(1), carry)

        WV = 32

        def wave(w, acc):
            def fire(i, _):
                r = gsel[pl.ds(w * WV + i, 16)][0]
                pltpu.async_copy(xf_hbm.at[g, r, :], rows_buf.at[i], sem)
                return 0
            lax.fori_loop(0, WV, fire, 0)
            pltpu.make_async_copy(xf_hbm.at[g, pl.ds(0, WV), :], rows_buf, sem).wait()

            def rsum(i, acc2):
                b0, b1, b2, b3 = acc2
                b0 = b0 + rows_buf[i, pl.ds(0, 16)]
                b1 = b1 + rows_buf[i, pl.ds(16, 16)]
                b2 = b2 + rows_buf[i, pl.ds(32, 16)]
                b3 = b3 + rows_buf[i, pl.ds(48, 16)]
                return (b0, b1, b2, b3)

            return lax.fori_loop(0, WV, rsum, acc)

        st0, st1, st2, st3 = lax.fori_loop(0, K // WV, wave, (zf, zf, zf, zf))
        vec64_b[pl.ds(0, 16)] = st0
        vec64_b[pl.ds(16, 16)] = st1
        vec64_b[pl.ds(32, 16)] = st2
        vec64_b[pl.ds(48, 16)] = st3
        pltpu.sync_copy(vec64_b, stop_hbm.at[pl.ds(g * DIM, DIM)])


def _k4_body(sums_ref, st_ref, w_ref, p_ref, out_ref):
    s2 = sums_ref[...]
    s = s2[:, :DIM] + s2[:, DIM:]
    st = st_ref[...]
    neg = s - st
    samples = jnp.concatenate([s, neg], axis=0)
    nrm = jnp.maximum(jnp.sqrt(jnp.sum(samples * samples, axis=1, keepdims=True)), 1e-8)
    sn = samples / nrm
    G = lax.dot_general(sn, sn, (((1,), (1,)), ((), ())))
    E = jnp.exp(G / TEMP)
    W = w_ref[...]
    P = p_ref[...]
    Ei = E[:_NPAIRS]
    Ej = E[L:L + _NPAIRS]
    Gp = jnp.sum(G[:_NPAIRS] * P, axis=1)
    Epos = jnp.sum(Ei * P, axis=1)
    den_i = Epos + jnp.sum(W * Ei, axis=1)
    den_j = Epos + jnp.sum(W * Ej, axis=1)
    loss = jnp.sum(jnp.log(den_i) + jnp.log(den_j) - 2.0 * Gp / TEMP)
    out_ref[...] = (loss / (_NPAIRS * 2)).reshape(1, 1)


@jax.jit
def kernel(I_embeddings):
    sums2, stop = pl.kernel(
        _sc_body,
        out_type=[
            jax.ShapeDtypeStruct((D * 2 * DIM,), jnp.float32),
            jax.ShapeDtypeStruct((D * DIM,), jnp.float32),
        ],
        mesh=plsc.VectorSubcoreMesh(core_axis_name="c", subcore_axis_name="s"),
        compiler_params=pltpu.CompilerParams(needs_layout_passes=False),
        scratch_types=[
            pltpu.VMEM((CH, DIM), jnp.float32),
            pltpu.VMEM((CH, DIM), jnp.float32),
            pltpu.VMEM((HN,), jnp.float32),
            pltpu.VMEM((CAND,), jnp.float32),
            pltpu.VMEM((CAND,), jnp.int32),
            pltpu.VMEM((2, CAND), jnp.float32),
            pltpu.VMEM((2, CAND), jnp.int32),
            pltpu.VMEM((CAND,), jnp.int32),
            pltpu.VMEM((32, DIM), jnp.float32),
            pltpu.VMEM((DIM,), jnp.float32),
            pltpu.VMEM((DIM,), jnp.float32),
            pltpu.VMEM_SHARED((16, CAND), jnp.float32),
            pltpu.VMEM_SHARED((16, CAND), jnp.int32),
            pltpu.SemaphoreType.DMA,
            pltpu.SemaphoreType.DMA,
        ],
    )(I_embeddings)

    loss = pl.pallas_call(
        _k4_body,
        grid=(1,),
        in_specs=[
            pl.BlockSpec((D, 2 * DIM), lambda _: (0, 0)),
            pl.BlockSpec((D, DIM), lambda _: (0, 0)),
            pl.BlockSpec((_NPAIRS, 2 * D), lambda _: (0, 0)),
            pl.BlockSpec((_NPAIRS, 2 * D), lambda _: (0, 0)),
        ],
        out_specs=pl.BlockSpec((1, 1), lambda _: (0, 0)),
        out_shape=jax.ShapeDtypeStruct((1, 1), jnp.float32),
    )(sums2.reshape(D, 2 * DIM), stop.reshape(D, DIM),
      jnp.asarray(_W_CONST), jnp.asarray(_P_CONST))

    return loss[0, 0]
